# baseline (speedup 1.0000x reference)
import functools

import jax
import jax.numpy as jnp
import numpy as np
from jax import lax
from jax.experimental import pallas as pl
from jax.experimental.pallas import tpu as pltpu
from jax.experimental.pallas import tpu_sc as plsc

VOCAB = 1000
BATCH = 16384
DIM_A = 32
DIM_B = 64
DIM_F = 128
MAX_NORM_A = float(np.sqrt(DIM_A))
MAX_NORM_B = float(np.sqrt(DIM_B))

_INFO = plsc.get_sparse_core_info()
_NC = _INFO.num_cores
_NS = _INFO.num_subcores
_NW = _NC * _NS
_BPW = BATCH // _NW
_CHUNK = 128
_NCHUNK = _BPW // _CHUNK


def _renorm_body(ta_ref, tb_ref, of_ref):
    a = ta_ref[...]
    na = jnp.sqrt(jnp.sum(a * a, axis=1, keepdims=True))
    sa = jnp.where(na > MAX_NORM_A, MAX_NORM_A / (na + 1e-7), 1.0)
    b = tb_ref[...]
    nb = jnp.sqrt(jnp.sum(b * b, axis=1, keepdims=True))
    sb = jnp.where(nb > MAX_NORM_B, MAX_NORM_B / (nb + 1e-7), 1.0)
    pad = jnp.zeros((VOCAB, DIM_F - DIM_A - DIM_B), jnp.float32)
    of_ref[...] = jnp.concatenate([a * sa, b * sb, pad], axis=1)


_renorm = pl.pallas_call(
    _renorm_body,
    out_shape=jax.ShapeDtypeStruct((VOCAB, DIM_F), jnp.float32),
)


@functools.partial(
    pl.kernel,
    mesh=plsc.VectorSubcoreMesh(core_axis_name="c", subcore_axis_name="s"),
    out_type=jax.ShapeDtypeStruct((BATCH, DIM_F), jnp.float32),
    scratch_types=[
        pltpu.VMEM((_BPW,), jnp.int32),
        pltpu.VMEM((_BPW, DIM_F), jnp.float32),
        pltpu.SemaphoreType.DMA,
    ],
    compiler_params=pltpu.CompilerParams(use_tc_tiling_on_sc=True),
)
def _gather(ids_hbm, tf_hbm, of_hbm, idx_v, rows_f, sem):
    wid = lax.axis_index("s") * _NC + lax.axis_index("c")
    base = wid * _BPW
    pltpu.sync_copy(ids_hbm.at[pl.ds(base, _BPW)], idx_v)
    copies = []
    for j in range(_NCHUNK):
        sl = pl.ds(j * _CHUNK, _CHUNK)
        copies.append(pltpu.async_copy(tf_hbm.at[idx_v.at[sl]], rows_f.at[sl], sem))
    for c in copies:
        c.wait()
    pltpu.sync_copy(rows_f, of_hbm.at[pl.ds(base, _BPW)])


_SPLIT_ROWS = 2048


def _split_body(f_ref, oa_ref, ob_ref):
    f = f_ref[...]
    oa_ref[...] = f[:, :DIM_A]
    ob_ref[...] = f[:, DIM_A:DIM_A + DIM_B]


_split = pl.pallas_call(
    _split_body,
    grid=(BATCH // _SPLIT_ROWS,),
    in_specs=[pl.BlockSpec((_SPLIT_ROWS, DIM_F), lambda i: (i, 0))],
    out_specs=(
        pl.BlockSpec((_SPLIT_ROWS, DIM_A), lambda i: (i, 0)),
        pl.BlockSpec((_SPLIT_ROWS, DIM_B), lambda i: (i, 0)),
    ),
    out_shape=(
        jax.ShapeDtypeStruct((BATCH, DIM_A), jnp.float32),
        jax.ShapeDtypeStruct((BATCH, DIM_B), jnp.float32),
    ),
)


@jax.jit
def kernel(ids, table_a, table_b):
    tf = _renorm(table_a, table_b)
    of = _gather(ids, tf)
    return _split(of)


# speedup vs baseline: 1.3941x; 1.3941x over previous
import functools

import jax
import jax.numpy as jnp
import numpy as np
from jax import lax
from jax.experimental import pallas as pl
from jax.experimental.pallas import tpu as pltpu
from jax.experimental.pallas import tpu_sc as plsc

VOCAB = 1000
VPAD = 1024
BATCH = 16384
DIM_A = 32
DIM_B = 64
MAX_NORM_A = float(np.sqrt(DIM_A))
MAX_NORM_B = float(np.sqrt(DIM_B))

_NBA = DIM_A // 8
_NBB = DIM_B // 8
_NTILE = BATCH // 128

_INFO = plsc.get_sparse_core_info()
_NC = _INFO.num_cores
_NS = _INFO.num_subcores
_NW = _NC * _NS
_SA = _NW // _NBA
_SB = _NW // _NBB
_TA = _NTILE // _SA
_TB = _NTILE // _SB


def _renorm_t_body(ta_ref, tb_ref, oa_ref, ob_ref):
    a = ta_ref[...]
    ap = jnp.concatenate([a, jnp.zeros((DIM_A, VPAD - VOCAB), jnp.float32)], axis=1)
    na = jnp.sqrt(jnp.sum(ap * ap, axis=0, keepdims=True))
    sa = jnp.where(na > MAX_NORM_A, MAX_NORM_A / (na + 1e-7), 1.0)
    A = ap * sa
    for blk in range(_NBA):
        for tc in range(VPAD // 128):
            oa_ref[blk, tc] = A[8 * blk:8 * blk + 8, 128 * tc:128 * tc + 128]
    b = tb_ref[...]
    bp = jnp.concatenate([b, jnp.zeros((DIM_B, VPAD - VOCAB), jnp.float32)], axis=1)
    nb = jnp.sqrt(jnp.sum(bp * bp, axis=0, keepdims=True))
    sb = jnp.where(nb > MAX_NORM_B, MAX_NORM_B / (nb + 1e-7), 1.0)
    B = bp * sb
    for blk in range(_NBB):
        for tc in range(VPAD // 128):
            ob_ref[blk, tc] = B[8 * blk:8 * blk + 8, 128 * tc:128 * tc + 128]


_renorm_t = pl.pallas_call(
    _renorm_t_body,
    out_shape=(
        jax.ShapeDtypeStruct((_NBA, VPAD // 128, 8, 128), jnp.float32),
        jax.ShapeDtypeStruct((_NBB, VPAD // 128, 8, 128), jnp.float32),
    ),
)


@functools.partial(
    pl.kernel,
    mesh=plsc.VectorSubcoreMesh(core_axis_name="c", subcore_axis_name="s"),
    out_type=(
        jax.ShapeDtypeStruct((_NBA, _NTILE, 8, 128), jnp.float32),
        jax.ShapeDtypeStruct((_NBB, _NTILE, 8, 128), jnp.float32),
    ),
    scratch_types=[
        pltpu.VMEM((BATCH // _SA,), jnp.int32),
        pltpu.VMEM((BATCH // _SB,), jnp.int32),
        pltpu.VMEM((VPAD // 128, 8, 128), jnp.float32),
        pltpu.VMEM((VPAD // 128, 8, 128), jnp.float32),
        pltpu.VMEM((_TA, 8, 128), jnp.float32),
        pltpu.VMEM((_TB, 8, 128), jnp.float32),
        pltpu.SemaphoreType.DMA,
    ],
    compiler_params=pltpu.CompilerParams(
        use_tc_tiling_on_sc=True, needs_layout_passes=False),
)
def _gather_t(ids_hbm, ta4_hbm, tb4_hbm, oa4_hbm, ob4_hbm,
              idxa, idxb, tba, tbb, osa, osb, sem):
    wid = lax.axis_index("s") * _NC + lax.axis_index("c")

    ablk = wid // _SA
    s = wid % _SA
    pltpu.sync_copy(ids_hbm.at[pl.ds(s * (BATCH // _SA), BATCH // _SA)], idxa)
    pltpu.sync_copy(ta4_hbm.at[ablk], tba)

    def body_a(kb, _):
        for q in range(8):
            v = idxa[pl.ds(kb * 128 + q * 16, 16)]
            hi = lax.shift_right_logical(v, 7)
            lo = lax.bitwise_and(v, 127)
            for r in range(8):
                rv = jnp.full((16,), r, jnp.int32)
                g = plsc.load_gather(tba, [hi, rv, lo])
                osa[kb, r, pl.ds(q * 16, 16)] = g
        return _

    lax.fori_loop(0, _TA, body_a, None)
    cp_a = pltpu.async_copy(osa, oa4_hbm.at[ablk, pl.ds(s * _TA, _TA)], sem)

    bblk = wid // _SB
    s2 = wid % _SB
    pltpu.sync_copy(ids_hbm.at[pl.ds(s2 * (BATCH // _SB), BATCH // _SB)], idxb)
    pltpu.sync_copy(tb4_hbm.at[bblk], tbb)

    def body_b(kb, _):
        for q in range(8):
            v = idxb[pl.ds(kb * 128 + q * 16, 16)]
            hi = lax.shift_right_logical(v, 7)
            lo = lax.bitwise_and(v, 127)
            for r in range(8):
                rv = jnp.full((16,), r, jnp.int32)
                g = plsc.load_gather(tbb, [hi, rv, lo])
                osb[kb, r, pl.ds(q * 16, 16)] = g
        return _

    lax.fori_loop(0, _TB, body_b, None)
    cp_b = pltpu.async_copy(osb, ob4_hbm.at[bblk, pl.ds(s2 * _TB, _TB)], sem)
    cp_a.wait()
    cp_b.wait()


@jax.jit
def kernel(ids, table_a, table_b):
    ta4, tb4 = _renorm_t(table_a.T, table_b.T)
    oa4, ob4 = _gather_t(ids, ta4, tb4)
    oa = oa4.transpose(0, 2, 1, 3).reshape(DIM_A, BATCH).T
    ob = ob4.transpose(0, 2, 1, 3).reshape(DIM_B, BATCH).T
    return oa, ob


# speedup vs baseline: 1.4816x; 1.0628x over previous
import functools

import jax
import jax.numpy as jnp
import numpy as np
from jax import lax
from jax.experimental import pallas as pl
from jax.experimental.pallas import tpu as pltpu
from jax.experimental.pallas import tpu_sc as plsc

VOCAB = 1000
VPAD = 1024
BATCH = 16384
DIM_A = 32
DIM_B = 64
MAX_NORM_A = float(np.sqrt(DIM_A))
MAX_NORM_B = float(np.sqrt(DIM_B))

_NBA = DIM_A // 8
_NBB = DIM_B // 8
_NTILE = BATCH // 128

_INFO = plsc.get_sparse_core_info()
_NC = _INFO.num_cores
_NS = _INFO.num_subcores
_NW = _NC * _NS
_SA = _NW // _NBA
_SB = _NW // _NBB
_TA = _NTILE // _SA
_TB = _NTILE // _SB
_BLK = 8 * VPAD


def _renorm_t_body(ta_ref, tb_ref, oa_ref, ob_ref):
    a = ta_ref[...]
    ap = jnp.concatenate([a, jnp.zeros((DIM_A, VPAD - VOCAB), jnp.float32)], axis=1)
    na = jnp.sqrt(jnp.sum(ap * ap, axis=0, keepdims=True))
    sa = jnp.where(na > MAX_NORM_A, MAX_NORM_A / (na + 1e-7), 1.0)
    A = ap * sa
    for blk in range(_NBA):
        for tc in range(VPAD // 128):
            base = blk * _BLK + tc * (8 * 128)
            oa_ref[pl.ds(base, 8 * 128)] = (
                A[8 * blk:8 * blk + 8, 128 * tc:128 * tc + 128].reshape(-1))
    b = tb_ref[...]
    bp = jnp.concatenate([b, jnp.zeros((DIM_B, VPAD - VOCAB), jnp.float32)], axis=1)
    nb = jnp.sqrt(jnp.sum(bp * bp, axis=0, keepdims=True))
    sb = jnp.where(nb > MAX_NORM_B, MAX_NORM_B / (nb + 1e-7), 1.0)
    B = bp * sb
    for blk in range(_NBB):
        for tc in range(VPAD // 128):
            base = blk * _BLK + tc * (8 * 128)
            ob_ref[pl.ds(base, 8 * 128)] = (
                B[8 * blk:8 * blk + 8, 128 * tc:128 * tc + 128].reshape(-1))


_renorm_t = pl.pallas_call(
    _renorm_t_body,
    out_shape=(
        jax.ShapeDtypeStruct((_NBA * _BLK,), jnp.float32),
        jax.ShapeDtypeStruct((_NBB * _BLK,), jnp.float32),
    ),
)


@functools.partial(
    pl.kernel,
    mesh=plsc.VectorSubcoreMesh(core_axis_name="c", subcore_axis_name="s"),
    out_type=(
        jax.ShapeDtypeStruct((_NBA * _NTILE * 8 * 128,), jnp.float32),
        jax.ShapeDtypeStruct((_NBB * _NTILE * 8 * 128,), jnp.float32),
    ),
    scratch_types=[
        pltpu.VMEM((BATCH // _SA,), jnp.int32),
        pltpu.VMEM((BATCH // _SB,), jnp.int32),
        pltpu.VMEM((_BLK,), jnp.float32),
        pltpu.VMEM((_BLK,), jnp.float32),
        pltpu.VMEM((_TA * 8 * 128,), jnp.float32),
        pltpu.VMEM((_TB * 8 * 128,), jnp.float32),
        pltpu.SemaphoreType.DMA,
    ],
    compiler_params=pltpu.CompilerParams(
        use_tc_tiling_on_sc=True, needs_layout_passes=False),
)
def _gather_t(ids_hbm, ta_hbm, tb_hbm, oa_hbm, ob_hbm,
              idxa, idxb, tba, tbb, osa, osb, sem):
    wid = lax.axis_index("s") * _NC + lax.axis_index("c")

    ablk = wid // _SA
    s = wid % _SA
    bblk = wid // _SB
    s2 = wid % _SB

    cps = [
        pltpu.async_copy(
            ids_hbm.at[pl.ds(s * (BATCH // _SA), BATCH // _SA)], idxa, sem),
        pltpu.async_copy(
            ids_hbm.at[pl.ds(s2 * (BATCH // _SB), BATCH // _SB)], idxb, sem),
        pltpu.async_copy(ta_hbm.at[pl.ds(ablk * _BLK, _BLK)], tba, sem),
        pltpu.async_copy(tb_hbm.at[pl.ds(bblk * _BLK, _BLK)], tbb, sem),
    ]
    for c in cps:
        c.wait()

    def body_a(kb, _):
        for q in range(8):
            v = idxa[pl.ds(kb * 128 + q * 16, 16)]
            fl = lax.shift_left(lax.shift_right_logical(v, 7), 10) + \
                lax.bitwise_and(v, 127)
            for r in range(8):
                g = plsc.load_gather(tba, [fl + (r * 128)])
                osa[pl.ds(kb * 1024 + r * 128 + q * 16, 16)] = g
        return _

    lax.fori_loop(0, _TA, body_a, None)
    cp_a = pltpu.async_copy(
        osa, oa_hbm.at[pl.ds(ablk * (_NTILE * 1024) + s * (_TA * 1024),
                             _TA * 1024)], sem)

    def body_b(kb, _):
        for q in range(8):
            v = idxb[pl.ds(kb * 128 + q * 16, 16)]
            fl = lax.shift_left(lax.shift_right_logical(v, 7), 10) + \
                lax.bitwise_and(v, 127)
            for r in range(8):
                g = plsc.load_gather(tbb, [fl + (r * 128)])
                osb[pl.ds(kb * 1024 + r * 128 + q * 16, 16)] = g
        return _

    lax.fori_loop(0, _TB, body_b, None)
    cp_b = pltpu.async_copy(
        osb, ob_hbm.at[pl.ds(bblk * (_NTILE * 1024) + s2 * (_TB * 1024),
                             _TB * 1024)], sem)
    cp_a.wait()
    cp_b.wait()


@jax.jit
def kernel(ids, table_a, table_b):
    ta1, tb1 = _renorm_t(table_a.T, table_b.T)
    oa1, ob1 = _gather_t(ids, ta1, tb1)
    oa = (oa1.reshape(_NBA, _NTILE, 8, 128).transpose(0, 2, 1, 3)
          .reshape(DIM_A, BATCH).T)
    ob = (ob1.reshape(_NBB, _NTILE, 8, 128).transpose(0, 2, 1, 3)
          .reshape(DIM_B, BATCH).T)
    return oa, ob


# speedup vs baseline: 1.9109x; 1.2898x over previous
import functools

import jax
import jax.numpy as jnp
import numpy as np
from jax import lax
from jax.experimental import pallas as pl
from jax.experimental.pallas import tpu as pltpu
from jax.experimental.pallas import tpu_sc as plsc

VOCAB = 1000
VPAD = 1024
BATCH = 16384
DIM_A = 32
DIM_B = 64
MAX_NORM_A = float(np.sqrt(DIM_A))
MAX_NORM_B = float(np.sqrt(DIM_B))

_NBA = DIM_A // 8
_NBB = DIM_B // 8
_NTILE = BATCH // 128

_INFO = plsc.get_sparse_core_info()
_NC = _INFO.num_cores
_NS = _INFO.num_subcores
_NW = _NC * _NS
_SA = _NW // _NBA
_SB = _NW // _NBB
_TA = _NTILE // _SA
_TB = _NTILE // _SB
_BLK = 8 * VPAD


def _renorm_t_body(ta_ref, tb_ref, oa_ref, ob_ref):
    a = ta_ref[...]
    ap = jnp.concatenate([a, jnp.zeros((DIM_A, VPAD - VOCAB), jnp.float32)], axis=1)
    na = jnp.sqrt(jnp.sum(ap * ap, axis=0, keepdims=True))
    sa = jnp.where(na > MAX_NORM_A, MAX_NORM_A / (na + 1e-7), 1.0)
    A = ap * sa
    for blk in range(_NBA):
        for tc in range(VPAD // 128):
            base = blk * _BLK + tc * (8 * 128)
            oa_ref[pl.ds(base, 8 * 128)] = (
                A[8 * blk:8 * blk + 8, 128 * tc:128 * tc + 128].reshape(-1))
    b = tb_ref[...]
    bp = jnp.concatenate([b, jnp.zeros((DIM_B, VPAD - VOCAB), jnp.float32)], axis=1)
    nb = jnp.sqrt(jnp.sum(bp * bp, axis=0, keepdims=True))
    sb = jnp.where(nb > MAX_NORM_B, MAX_NORM_B / (nb + 1e-7), 1.0)
    B = bp * sb
    for blk in range(_NBB):
        for tc in range(VPAD // 128):
            base = blk * _BLK + tc * (8 * 128)
            ob_ref[pl.ds(base, 8 * 128)] = (
                B[8 * blk:8 * blk + 8, 128 * tc:128 * tc + 128].reshape(-1))


_renorm_t = pl.pallas_call(
    _renorm_t_body,
    out_shape=(
        jax.ShapeDtypeStruct((_NBA * _BLK,), jnp.float32),
        jax.ShapeDtypeStruct((_NBB * _BLK,), jnp.float32),
    ),
)


@functools.partial(
    pl.kernel,
    mesh=plsc.VectorSubcoreMesh(core_axis_name="c", subcore_axis_name="s"),
    out_type=(
        jax.ShapeDtypeStruct((_NBA * _NTILE * 8 * 128,), jnp.float32),
        jax.ShapeDtypeStruct((_NBB * _NTILE * 8 * 128,), jnp.float32),
    ),
    scratch_types=[
        pltpu.VMEM((BATCH // _SA,), jnp.int32),
        pltpu.VMEM((BATCH // _SB,), jnp.int32),
        pltpu.VMEM((_BLK,), jnp.float32),
        pltpu.VMEM((_BLK,), jnp.float32),
        pltpu.VMEM((_TA * 8 * 128,), jnp.float32),
        pltpu.VMEM((_TB * 8 * 128,), jnp.float32),
        pltpu.SemaphoreType.DMA,
    ],
    compiler_params=pltpu.CompilerParams(
        use_tc_tiling_on_sc=True, needs_layout_passes=False,
        disable_bounds_checks=True),
)
def _gather_t(ids_hbm, ta_hbm, tb_hbm, oa_hbm, ob_hbm,
              idxa, idxb, tba, tbb, osa, osb, sem):
    wid = lax.axis_index("s") * _NC + lax.axis_index("c")

    ablk = wid // _SA
    s = wid % _SA
    bblk = wid // _SB
    s2 = wid % _SB

    cps = [
        pltpu.async_copy(
            ids_hbm.at[pl.ds(s * (BATCH // _SA), BATCH // _SA)], idxa, sem),
        pltpu.async_copy(
            ids_hbm.at[pl.ds(s2 * (BATCH // _SB), BATCH // _SB)], idxb, sem),
        pltpu.async_copy(ta_hbm.at[pl.ds(ablk * _BLK, _BLK)], tba, sem),
        pltpu.async_copy(tb_hbm.at[pl.ds(bblk * _BLK, _BLK)], tbb, sem),
    ]
    for c in cps:
        c.wait()

    @plsc.parallel_loop(0, _TA)
    def body_a(kb):
        for q in range(8):
            v = idxa[pl.ds(kb * 128 + q * 16, 16)]
            fl = lax.shift_left(lax.shift_right_logical(v, 7), 10) + \
                lax.bitwise_and(v, 127)
            for r in range(8):
                g = plsc.load_gather(tba, [fl + (r * 128)])
                osa[pl.ds(kb * 1024 + r * 128 + q * 16, 16)] = g
    cp_a = pltpu.async_copy(
        osa, oa_hbm.at[pl.ds(ablk * (_NTILE * 1024) + s * (_TA * 1024),
                             _TA * 1024)], sem)

    @plsc.parallel_loop(0, _TB)
    def body_b(kb):
        for q in range(8):
            v = idxb[pl.ds(kb * 128 + q * 16, 16)]
            fl = lax.shift_left(lax.shift_right_logical(v, 7), 10) + \
                lax.bitwise_and(v, 127)
            for r in range(8):
                g = plsc.load_gather(tbb, [fl + (r * 128)])
                osb[pl.ds(kb * 1024 + r * 128 + q * 16, 16)] = g
    cp_b = pltpu.async_copy(
        osb, ob_hbm.at[pl.ds(bblk * (_NTILE * 1024) + s2 * (_TB * 1024),
                             _TB * 1024)], sem)
    cp_a.wait()
    cp_b.wait()


@jax.jit
def kernel(ids, table_a, table_b):
    ta1, tb1 = _renorm_t(table_a.T, table_b.T)
    oa1, ob1 = _gather_t(ids, ta1, tb1)
    oa = (oa1.reshape(_NBA, _NTILE, 8, 128).transpose(0, 2, 1, 3)
          .reshape(DIM_A, BATCH).T)
    ob = (ob1.reshape(_NBB, _NTILE, 8, 128).transpose(0, 2, 1, 3)
          .reshape(DIM_B, BATCH).T)
    return oa, ob
